# Initial kernel scaffold; baseline (speedup 1.0000x reference)
import functools

import jax
import jax.numpy as jnp
from jax import lax
from jax.experimental import pallas as pl
from jax.experimental.pallas import tpu as pltpu
from jax.experimental.pallas import tpu_sc as plsc

N = 10000
E = 320000
D = 128
H = 32

NC = 2
NS = 16
NW = NC * NS
CHUNK = 128
NCHUNK = 80
EPT = CHUNK * NCHUNK
E_PAD = EPT * NW
N_PAD = 10240
RPT = N_PAD // NS

_mesh = plsc.VectorSubcoreMesh(core_axis_name="c", subcore_axis_name="s")


@functools.partial(
    pl.kernel,
    out_type=jax.ShapeDtypeStruct((NC, N_PAD, 16), jnp.float32),
    mesh=_mesh,
    scratch_types=[
        pltpu.VMEM((NCHUNK, CHUNK), jnp.int32),
        pltpu.VMEM((CHUNK, 16), jnp.float32),
        pltpu.VMEM_SHARED((N_PAD, 16), jnp.float32),
    ],
)
def _deg_kernel(dst_hbm, out_hbm, dst_l, buf, deg_sh):
    cid = lax.axis_index("c")
    sid = lax.axis_index("s")
    wid = cid * NS + sid

    @pl.loop(0, CHUNK)
    def _(i):
        buf[i, :] = jnp.zeros((16,), jnp.float32)

    @pl.loop(0, RPT // CHUNK)
    def _(k):
        pltpu.sync_copy(buf, deg_sh.at[pl.ds(sid * RPT + k * CHUNK, CHUNK)])

    @pl.loop(0, CHUNK)
    def _(i):
        buf[i, :] = jnp.ones((16,), jnp.float32)

    plsc.subcore_barrier()
    pltpu.sync_copy(dst_hbm.at[wid], dst_l)

    @pl.loop(0, NCHUNK)
    def _(c):
        pltpu.sync_copy(buf, deg_sh.at[dst_l.at[c]], add=True)

    plsc.subcore_barrier()
    pltpu.sync_copy(deg_sh.at[pl.ds(sid * RPT, RPT)],
                    out_hbm.at[cid, pl.ds(sid * RPT, RPT)])


@functools.partial(
    pl.kernel,
    out_type=jax.ShapeDtypeStruct((NC, N_PAD, D), jnp.float32),
    mesh=_mesh,
    scratch_types=[
        pltpu.VMEM((EPT,), jnp.int32),
        pltpu.VMEM((NCHUNK, CHUNK), jnp.int32),
        pltpu.VMEM((CHUNK, D), jnp.float32),
        pltpu.VMEM((CHUNK, D), jnp.float32),
        pltpu.VMEM_SHARED((N_PAD, D), jnp.float32),
        pltpu.SemaphoreType.DMA,
        pltpu.SemaphoreType.DMA,
    ],
)
def _spmm_kernel(xs_hbm, src_hbm, dst_hbm, out_hbm,
                 src_l, dst_l, rows0, rows1, z_sh, sem0, sem1):
    cid = lax.axis_index("c")
    sid = lax.axis_index("s")
    wid = cid * NS + sid

    @pl.loop(0, CHUNK)
    def _(i):
        @pl.loop(0, D // 16)
        def _(j):
            rows0[i, pl.ds(j * 16, 16)] = jnp.zeros((16,), jnp.float32)

    @pl.loop(0, RPT // CHUNK)
    def _(k):
        pltpu.sync_copy(rows0, z_sh.at[pl.ds(sid * RPT + k * CHUNK, CHUNK)])

    plsc.subcore_barrier()

    pltpu.sync_copy(src_hbm.at[pl.ds(wid * EPT, EPT)], src_l)
    pltpu.sync_copy(dst_hbm.at[wid], dst_l)

    def gather(c, rows, sem):
        return pltpu.make_async_copy(
            xs_hbm.at[src_l.at[pl.ds(c * CHUNK, CHUNK)]], rows, sem)

    gather(0, rows0, sem0).start()
    gather(1, rows1, sem1).start()

    @pl.loop(0, NCHUNK // 2)
    def _(g):
        c0 = g * 2
        c1 = c0 + 1
        gather(c0, rows0, sem0).wait()
        pltpu.sync_copy(rows0, z_sh.at[dst_l.at[c0]], add=True)

        @pl.when(c0 + 2 < NCHUNK)
        def _():
            gather(c0 + 2, rows0, sem0).start()

        gather(c1, rows1, sem1).wait()
        pltpu.sync_copy(rows1, z_sh.at[dst_l.at[c1]], add=True)

        @pl.when(c1 + 2 < NCHUNK)
        def _():
            gather(c1 + 2, rows1, sem1).start()

    plsc.subcore_barrier()
    pltpu.sync_copy(z_sh.at[pl.ds(sid * RPT, RPT)],
                    out_hbm.at[cid, pl.ds(sid * RPT, RPT)])


BR = 400
GRID = N // BR


def _prep_body(x_ref, degp_ref, xs_ref, dinv_ref):
    deg = degp_ref[0, :, 0:1] + degp_ref[1, :, 0:1] + 1.0
    dinv = lax.rsqrt(deg)
    dinv_ref[...] = dinv
    xs_ref[...] = x_ref[...] * dinv


_prep = pl.pallas_call(
    _prep_body,
    grid=(GRID,),
    in_specs=[
        pl.BlockSpec((BR, D), lambda i: (i, 0)),
        pl.BlockSpec((NC, BR, 16), lambda i: (0, i, 0)),
    ],
    out_specs=[
        pl.BlockSpec((BR, D), lambda i: (i, 0)),
        pl.BlockSpec((BR, 1), lambda i: (i, 0)),
    ],
    out_shape=[
        jax.ShapeDtypeStruct((N, D), jnp.float32),
        jax.ShapeDtypeStruct((N, 1), jnp.float32),
    ],
)


def _epi_body(zp, xs, dinv, x,
              aw, ab, a1w, a1b, a2w, a2b, a3w, a3b,
              cw, cb, c1w, c1b, c2w, c2b, c3w, c3b,
              conc_ref, val_ref, csum):
    i = pl.program_id(0)
    f32 = jnp.float32
    z = zp[0] + zp[1]
    y = (z + xs[...]) * dinv[...]
    a = jnp.maximum(jnp.dot(y, aw[...], preferred_element_type=f32) + ab[...], 0.0)
    a = a + x[...]
    a = jnp.maximum(jnp.dot(a, a1w[...], preferred_element_type=f32) + a1b[...], 0.0)
    a = jnp.maximum(jnp.dot(a, a2w[...], preferred_element_type=f32) + a2b[...], 0.0)
    a3 = jnp.dot(a, a3w[...], preferred_element_type=f32) + a3b[...]
    conc_ref[...] = jax.nn.softplus(a3) + 1e-20

    c = jnp.maximum(jnp.dot(y, cw[...], preferred_element_type=f32) + cb[...], 0.0)
    c = c + x[...]

    @pl.when(i == 0)
    def _():
        csum[...] = jnp.zeros_like(csum)

    csum[...] += jnp.sum(c, axis=0, keepdims=True)

    @pl.when(i == pl.num_programs(0) - 1)
    def _():
        cv = csum[...]
        h = jnp.maximum(jnp.dot(cv, c1w[...], preferred_element_type=f32) + c1b[...], 0.0)
        h = jnp.maximum(jnp.dot(h, c2w[...], preferred_element_type=f32) + c2b[...], 0.0)
        val_ref[...] = jnp.dot(h, c3w[...], preferred_element_type=f32) + c3b[...]


def _full(shape):
    return pl.BlockSpec(shape, lambda i: tuple(0 for _ in shape))


_epi = pl.pallas_call(
    _epi_body,
    grid=(GRID,),
    in_specs=[
        pl.BlockSpec((NC, BR, D), lambda i: (0, i, 0)),
        pl.BlockSpec((BR, D), lambda i: (i, 0)),
        pl.BlockSpec((BR, 1), lambda i: (i, 0)),
        pl.BlockSpec((BR, D), lambda i: (i, 0)),
        _full((D, D)), _full((1, D)),
        _full((D, H)), _full((1, H)),
        _full((H, H)), _full((1, H)),
        _full((H, 1)), _full((1, 1)),
        _full((D, D)), _full((1, D)),
        _full((D, H)), _full((1, H)),
        _full((H, H)), _full((1, H)),
        _full((H, 1)), _full((1, 1)),
    ],
    out_specs=[
        pl.BlockSpec((BR, 1), lambda i: (i, 0)),
        pl.BlockSpec((1, 1), lambda i: (0, 0)),
    ],
    out_shape=[
        jax.ShapeDtypeStruct((N, 1), jnp.float32),
        jax.ShapeDtypeStruct((1, 1), jnp.float32),
    ],
    scratch_shapes=[pltpu.VMEM((1, D), jnp.float32)],
)


def kernel(x, edge_index,
           actor_conv_w, actor_conv_b, actor_l1_w, actor_l1_b,
           actor_l2_w, actor_l2_b, actor_l3_w, actor_l3_b,
           critic_conv_w, critic_conv_b, critic_l1_w, critic_l1_b,
           critic_l2_w, critic_l2_b, critic_l3_w, critic_l3_b):
    pad = E_PAD - E
    src_p = jnp.concatenate(
        [edge_index[0], jnp.zeros((pad,), edge_index.dtype)])
    dst3 = jnp.concatenate(
        [edge_index[1], jnp.full((pad,), N, edge_index.dtype)]
    ).reshape(NW, NCHUNK, CHUNK)

    degp = _deg_kernel(dst3)
    xs, dinv = _prep(x, degp)
    zp = _spmm_kernel(xs, src_p, dst3)

    conc2d, val = _epi(
        zp, xs, dinv, x,
        actor_conv_w, actor_conv_b.reshape(1, D),
        actor_l1_w, actor_l1_b.reshape(1, H),
        actor_l2_w, actor_l2_b.reshape(1, H),
        actor_l3_w, actor_l3_b.reshape(1, 1),
        critic_conv_w, critic_conv_b.reshape(1, D),
        critic_l1_w, critic_l1_b.reshape(1, H),
        critic_l2_w, critic_l2_b.reshape(1, H),
        critic_l3_w, critic_l3_b.reshape(1, 1),
    )
    return conc2d.reshape(-1), val.reshape(-1)


# speedup vs baseline: 16.0261x; 16.0261x over previous
import dataclasses
import functools

import jax
import jax.numpy as jnp
from jax import lax
from jax.experimental import pallas as pl
from jax.experimental.pallas import tpu as pltpu
from jax.experimental.pallas import tpu_sc as plsc

N = 10000
E = 320000
D = 128
H = 32

NC = 2
NS = 16
NW = NC * NS
CHUNK = 128
NCHUNK = 80
EPT = CHUNK * NCHUNK
E_PAD = EPT * NW
N_PAD = 10240
RPT = N_PAD // NS

_mesh = plsc.VectorSubcoreMesh(core_axis_name="c", subcore_axis_name="s")

_cp = pltpu.CompilerParams()
if "needs_layout_passes" in pltpu.CompilerParams.__dataclass_fields__:
    _cp = dataclasses.replace(_cp, needs_layout_passes=False)

HR = N_PAD // 128


@functools.partial(
    pl.kernel,
    out_type=jax.ShapeDtypeStruct((NC, HR, 128), jnp.float32),
    mesh=_mesh,
    scratch_types=[
        pltpu.VMEM((CHUNK,), jnp.int32),
        pltpu.VMEM((CHUNK,), jnp.int32),
        pltpu.VMEM((HR, 128), jnp.float32),
        pltpu.VMEM((HR,), jnp.int32),
        pltpu.VMEM_SHARED((HR, 128), jnp.float32),
        pltpu.SemaphoreType.DMA,
        pltpu.SemaphoreType.DMA,
    ],
    compiler_params=_cp,
)
def _deg_kernel(dst_hbm, out_hbm, di0, di1, hist, rowidx, deg_sh, si0, si1):
    cid = lax.axis_index("c")
    sid = lax.axis_index("s")
    wid = cid * NS + sid
    i16 = lax.iota(jnp.int32, 16)
    ones = jnp.ones((16,), jnp.float32)

    @pl.loop(0, HR)
    def _(i):
        @pl.loop(0, 8)
        def _(j):
            hist[i, pl.ds(j * 16, 16)] = jnp.zeros((16,), jnp.float32)

    @pl.loop(0, HR // 16)
    def _(k):
        rowidx[pl.ds(k * 16, 16)] = i16 + k * 16

    @pl.when(sid == 0)
    def _():
        pltpu.sync_copy(hist, deg_sh)

    def idx_dma(c, dv, sem):
        return pltpu.make_async_copy(dst_hbm.at[wid, c], dv, sem)

    idx_dma(0, di0, si0).start()
    idx_dma(1, di1, si1).start()

    def accum(dv):
        @pl.loop(0, CHUNK // 16)
        def _(k):
            ids = dv[pl.ds(k * 16, 16)]
            row = lax.shift_right_logical(ids, 7)
            col = lax.bitwise_and(ids, 127)
            plsc.addupdate_scatter(hist, [row, col], ones)

    @pl.loop(0, NCHUNK // 2 - 1)
    def _(g):
        c0 = g * 2
        idx_dma(c0, di0, si0).wait()
        accum(di0)
        idx_dma(c0 + 2, di0, si0).start()
        idx_dma(c0 + 1, di1, si1).wait()
        accum(di1)
        idx_dma(c0 + 3, di1, si1).start()

    idx_dma(NCHUNK - 2, di0, si0).wait()
    accum(di0)
    idx_dma(NCHUNK - 1, di1, si1).wait()
    accum(di1)

    plsc.subcore_barrier()
    pltpu.sync_copy(hist, deg_sh.at[rowidx], add=True)
    plsc.subcore_barrier()

    @pl.when(sid < 10)
    def _():
        pltpu.sync_copy(deg_sh.at[pl.ds(sid * 8, 8)],
                        out_hbm.at[cid, pl.ds(sid * 8, 8)])


@functools.partial(
    pl.kernel,
    out_type=jax.ShapeDtypeStruct((NC, N_PAD, D), jnp.float32),
    mesh=_mesh,
    scratch_types=[
        pltpu.VMEM((CHUNK,), jnp.int32),
        pltpu.VMEM((CHUNK,), jnp.int32),
        pltpu.VMEM((CHUNK,), jnp.int32),
        pltpu.VMEM((CHUNK,), jnp.int32),
        pltpu.VMEM((CHUNK, D), jnp.float32),
        pltpu.VMEM((CHUNK, D), jnp.float32),
        pltpu.VMEM_SHARED((N_PAD, D), jnp.float32),
        pltpu.SemaphoreType.DMA,
        pltpu.SemaphoreType.DMA,
        pltpu.SemaphoreType.DMA,
        pltpu.SemaphoreType.DMA,
        pltpu.SemaphoreType.DMA,
        pltpu.SemaphoreType.DMA,
    ],
)
def _spmm_kernel(xs_hbm, src_hbm, dst_hbm, out_hbm,
                 s0, s1, d0, d1, rows0, rows1, z_sh,
                 ss0, ss1, ds0, ds1, gs0, gs1):
    cid = lax.axis_index("c")
    sid = lax.axis_index("s")
    wid = cid * NS + sid

    @pl.loop(0, CHUNK)
    def _(i):
        @pl.loop(0, D // 16)
        def _(j):
            rows0[i, pl.ds(j * 16, 16)] = jnp.zeros((16,), jnp.float32)

    @pl.loop(0, RPT // CHUNK)
    def _(k):
        pltpu.sync_copy(rows0, z_sh.at[pl.ds(sid * RPT + k * CHUNK, CHUNK)])

    plsc.subcore_barrier()

    def sidx(c, dv, sem):
        return pltpu.make_async_copy(src_hbm.at[wid, c], dv, sem)

    def didx(c, dv, sem):
        return pltpu.make_async_copy(dst_hbm.at[wid, c], dv, sem)

    def gath(sv, rows, sem):
        return pltpu.make_async_copy(xs_hbm.at[sv], rows, sem)

    sidx(0, s0, ss0).start()
    didx(0, d0, ds0).start()
    sidx(1, s1, ss1).start()
    didx(1, d1, ds1).start()
    sidx(0, s0, ss0).wait()
    gath(s0, rows0, gs0).start()
    sidx(1, s1, ss1).wait()
    gath(s1, rows1, gs1).start()

    @pl.loop(0, NCHUNK // 2 - 1)
    def _(g):
        c0 = g * 2
        gath(s0, rows0, gs0).wait()
        didx(c0, d0, ds0).wait()
        pltpu.sync_copy(rows0, z_sh.at[d0], add=True)
        sidx(c0 + 2, s0, ss0).start()
        didx(c0 + 2, d0, ds0).start()

        gath(s1, rows1, gs1).wait()
        didx(c0 + 1, d1, ds1).wait()
        pltpu.sync_copy(rows1, z_sh.at[d1], add=True)
        sidx(c0 + 3, s1, ss1).start()
        didx(c0 + 3, d1, ds1).start()

        sidx(c0 + 2, s0, ss0).wait()
        gath(s0, rows0, gs0).start()
        sidx(c0 + 3, s1, ss1).wait()
        gath(s1, rows1, gs1).start()

    gath(s0, rows0, gs0).wait()
    didx(NCHUNK - 2, d0, ds0).wait()
    pltpu.sync_copy(rows0, z_sh.at[d0], add=True)
    gath(s1, rows1, gs1).wait()
    didx(NCHUNK - 1, d1, ds1).wait()
    pltpu.sync_copy(rows1, z_sh.at[d1], add=True)

    plsc.subcore_barrier()
    pltpu.sync_copy(z_sh.at[pl.ds(sid * RPT, RPT)],
                    out_hbm.at[cid, pl.ds(sid * RPT, RPT)])


BR = 400
GRID = N // BR


def _prep_body(x_ref, degp_ref, xs_ref, dinv_ref):
    deg = degp_ref[0] + degp_ref[1] + 1.0
    dinv = lax.rsqrt(deg)
    dinv_ref[...] = dinv
    xs_ref[...] = x_ref[...] * dinv


_prep = pl.pallas_call(
    _prep_body,
    grid=(GRID,),
    in_specs=[
        pl.BlockSpec((BR, D), lambda i: (i, 0)),
        pl.BlockSpec((NC, BR, 1), lambda i: (0, i, 0)),
    ],
    out_specs=[
        pl.BlockSpec((BR, D), lambda i: (i, 0)),
        pl.BlockSpec((BR, 1), lambda i: (i, 0)),
    ],
    out_shape=[
        jax.ShapeDtypeStruct((N, D), jnp.float32),
        jax.ShapeDtypeStruct((N, 1), jnp.float32),
    ],
)


def _epi_body(zp, xs, dinv, x,
              aw, ab, a1w, a1b, a2w, a2b, a3w, a3b,
              cw, cb, c1w, c1b, c2w, c2b, c3w, c3b,
              conc_ref, val_ref, csum):
    i = pl.program_id(0)
    f32 = jnp.float32
    z = zp[0] + zp[1]
    y = (z + xs[...]) * dinv[...]
    a = jnp.maximum(jnp.dot(y, aw[...], preferred_element_type=f32) + ab[...], 0.0)
    a = a + x[...]
    a = jnp.maximum(jnp.dot(a, a1w[...], preferred_element_type=f32) + a1b[...], 0.0)
    a = jnp.maximum(jnp.dot(a, a2w[...], preferred_element_type=f32) + a2b[...], 0.0)
    a3 = jnp.dot(a, a3w[...], preferred_element_type=f32) + a3b[...]
    conc_ref[...] = jax.nn.softplus(a3) + 1e-20

    c = jnp.maximum(jnp.dot(y, cw[...], preferred_element_type=f32) + cb[...], 0.0)
    c = c + x[...]

    @pl.when(i == 0)
    def _():
        csum[...] = jnp.zeros_like(csum)

    csum[...] += jnp.sum(c, axis=0, keepdims=True)

    @pl.when(i == pl.num_programs(0) - 1)
    def _():
        cv = csum[...]
        h = jnp.maximum(jnp.dot(cv, c1w[...], preferred_element_type=f32) + c1b[...], 0.0)
        h = jnp.maximum(jnp.dot(h, c2w[...], preferred_element_type=f32) + c2b[...], 0.0)
        val_ref[...] = jnp.dot(h, c3w[...], preferred_element_type=f32) + c3b[...]


def _full(shape):
    return pl.BlockSpec(shape, lambda i: tuple(0 for _ in shape))


_epi = pl.pallas_call(
    _epi_body,
    grid=(GRID,),
    in_specs=[
        pl.BlockSpec((NC, BR, D), lambda i: (0, i, 0)),
        pl.BlockSpec((BR, D), lambda i: (i, 0)),
        pl.BlockSpec((BR, 1), lambda i: (i, 0)),
        pl.BlockSpec((BR, D), lambda i: (i, 0)),
        _full((D, D)), _full((1, D)),
        _full((D, H)), _full((1, H)),
        _full((H, H)), _full((1, H)),
        _full((H, 1)), _full((1, 1)),
        _full((D, D)), _full((1, D)),
        _full((D, H)), _full((1, H)),
        _full((H, H)), _full((1, H)),
        _full((H, 1)), _full((1, 1)),
    ],
    out_specs=[
        pl.BlockSpec((BR, 1), lambda i: (i, 0)),
        pl.BlockSpec((1, 1), lambda i: (0, 0)),
    ],
    out_shape=[
        jax.ShapeDtypeStruct((N, 1), jnp.float32),
        jax.ShapeDtypeStruct((1, 1), jnp.float32),
    ],
    scratch_shapes=[pltpu.VMEM((1, D), jnp.float32)],
)


def kernel(x, edge_index,
           actor_conv_w, actor_conv_b, actor_l1_w, actor_l1_b,
           actor_l2_w, actor_l2_b, actor_l3_w, actor_l3_b,
           critic_conv_w, critic_conv_b, critic_l1_w, critic_l1_b,
           critic_l2_w, critic_l2_b, critic_l3_w, critic_l3_b):
    pad = E_PAD - E
    src3 = jnp.concatenate(
        [edge_index[0], jnp.zeros((pad,), edge_index.dtype)]
    ).reshape(NW, NCHUNK, CHUNK)
    dst3 = jnp.concatenate(
        [edge_index[1], jnp.full((pad,), N, edge_index.dtype)]
    ).reshape(NW, NCHUNK, CHUNK)

    degp = _deg_kernel(dst3).reshape(NC, N_PAD, 1)
    xs, dinv = _prep(x, degp)
    zp = _spmm_kernel(xs, src3, dst3)

    conc2d, val = _epi(
        zp, xs, dinv, x,
        actor_conv_w, actor_conv_b.reshape(1, D),
        actor_l1_w, actor_l1_b.reshape(1, H),
        actor_l2_w, actor_l2_b.reshape(1, H),
        actor_l3_w, actor_l3_b.reshape(1, 1),
        critic_conv_w, critic_conv_b.reshape(1, D),
        critic_l1_w, critic_l1_b.reshape(1, H),
        critic_l2_w, critic_l2_b.reshape(1, H),
        critic_l3_w, critic_l3_b.reshape(1, 1),
    )
    return conc2d.reshape(-1), val.reshape(-1)


# speedup vs baseline: 35.6064x; 2.2218x over previous
import dataclasses
import functools

import jax
import jax.numpy as jnp
from jax import lax
from jax.experimental import pallas as pl
from jax.experimental.pallas import tpu as pltpu
from jax.experimental.pallas import tpu_sc as plsc

N = 10000
E = 320000
D = 128
H = 32

NC = 2
NS = 16
NW = NC * NS
CHUNK = 128
NCHUNK = 80
EPT = CHUNK * NCHUNK
E_PAD = EPT * NW
N_PAD = 10240
RPT = N_PAD // NS

_mesh = plsc.VectorSubcoreMesh(core_axis_name="c", subcore_axis_name="s")

_cp = pltpu.CompilerParams()
if "needs_layout_passes" in pltpu.CompilerParams.__dataclass_fields__:
    _cp = dataclasses.replace(_cp, needs_layout_passes=False)

HR = N_PAD // 128


@functools.partial(
    pl.kernel,
    out_type=jax.ShapeDtypeStruct((NC, HR, 128), jnp.float32),
    mesh=_mesh,
    scratch_types=[
        pltpu.VMEM((CHUNK,), jnp.int32),
        pltpu.VMEM((CHUNK,), jnp.int32),
        pltpu.VMEM((HR, 128), jnp.float32),
        pltpu.VMEM((HR,), jnp.int32),
        pltpu.VMEM_SHARED((HR, 128), jnp.float32),
        pltpu.SemaphoreType.DMA,
        pltpu.SemaphoreType.DMA,
    ],
    compiler_params=_cp,
)
def _deg_kernel(dst_hbm, out_hbm, di0, di1, hist, rowidx, deg_sh, si0, si1):
    cid = lax.axis_index("c")
    sid = lax.axis_index("s")
    wid = cid * NS + sid
    i16 = lax.iota(jnp.int32, 16)
    ones = jnp.ones((16,), jnp.float32)

    @pl.loop(0, HR)
    def _(i):
        @pl.loop(0, 8)
        def _(j):
            hist[i, pl.ds(j * 16, 16)] = jnp.zeros((16,), jnp.float32)

    @pl.loop(0, HR // 16)
    def _(k):
        rowidx[pl.ds(k * 16, 16)] = i16 + k * 16

    @pl.when(sid == 0)
    def _():
        pltpu.sync_copy(hist, deg_sh)

    def idx_dma(c, dv, sem):
        return pltpu.make_async_copy(dst_hbm.at[wid, c], dv, sem)

    idx_dma(0, di0, si0).start()
    idx_dma(1, di1, si1).start()

    def accum(dv):
        @pl.loop(0, CHUNK // 16)
        def _(k):
            ids = dv[pl.ds(k * 16, 16)]
            row = lax.shift_right_logical(ids, 7)
            col = lax.bitwise_and(ids, 127)
            plsc.addupdate_scatter(hist, [row, col], ones)

    @pl.loop(0, NCHUNK // 2 - 1)
    def _(g):
        c0 = g * 2
        idx_dma(c0, di0, si0).wait()
        accum(di0)
        idx_dma(c0 + 2, di0, si0).start()
        idx_dma(c0 + 1, di1, si1).wait()
        accum(di1)
        idx_dma(c0 + 3, di1, si1).start()

    idx_dma(NCHUNK - 2, di0, si0).wait()
    accum(di0)
    idx_dma(NCHUNK - 1, di1, si1).wait()
    accum(di1)

    plsc.subcore_barrier()
    pltpu.sync_copy(hist, deg_sh.at[rowidx], add=True)
    plsc.subcore_barrier()

    @pl.when(sid < 10)
    def _():
        pltpu.sync_copy(deg_sh.at[pl.ds(sid * 8, 8)],
                        out_hbm.at[cid, pl.ds(sid * 8, 8)])


@functools.partial(
    pl.kernel,
    out_type=jax.ShapeDtypeStruct((NC, N_PAD, D), jnp.float32),
    mesh=_mesh,
    scratch_types=[
        pltpu.VMEM((CHUNK,), jnp.int32),
        pltpu.VMEM((CHUNK,), jnp.int32),
        pltpu.VMEM((CHUNK,), jnp.int32),
        pltpu.VMEM((CHUNK,), jnp.int32),
        pltpu.VMEM((CHUNK, D), jnp.float32),
        pltpu.VMEM((CHUNK, D), jnp.float32),
        pltpu.VMEM_SHARED((N_PAD, D), jnp.float32),
        pltpu.SemaphoreType.DMA,
        pltpu.SemaphoreType.DMA,
        pltpu.SemaphoreType.DMA,
        pltpu.SemaphoreType.DMA,
        pltpu.SemaphoreType.DMA,
        pltpu.SemaphoreType.DMA,
    ],
)
def _spmm_kernel(xs_hbm, src_hbm, dst_hbm, out_hbm,
                 s0, s1, d0, d1, rows0, rows1, z_sh,
                 ss0, ss1, ds0, ds1, gs0, gs1):
    cid = lax.axis_index("c")
    sid = lax.axis_index("s")
    wid = cid * NS + sid

    @pl.loop(0, CHUNK)
    def _(i):
        @pl.loop(0, D // 16)
        def _(j):
            rows0[i, pl.ds(j * 16, 16)] = jnp.zeros((16,), jnp.float32)

    @pl.loop(0, RPT // CHUNK)
    def _(k):
        pltpu.sync_copy(rows0, z_sh.at[pl.ds(sid * RPT + k * CHUNK, CHUNK)])

    plsc.subcore_barrier()

    def sidx(c, dv, sem):
        return pltpu.make_async_copy(src_hbm.at[wid, c], dv, sem)

    def didx(c, dv, sem):
        return pltpu.make_async_copy(dst_hbm.at[wid, c], dv, sem)

    def gath(sv, rows, sem):
        return pltpu.make_async_copy(xs_hbm.at[sv], rows, sem)

    sidx(0, s0, ss0).start()
    didx(0, d0, ds0).start()
    sidx(1, s1, ss1).start()
    didx(1, d1, ds1).start()
    sidx(0, s0, ss0).wait()
    gath(s0, rows0, gs0).start()
    sidx(1, s1, ss1).wait()
    gath(s1, rows1, gs1).start()

    @pl.loop(0, NCHUNK // 2 - 1)
    def _(g):
        c0 = g * 2
        gath(s0, rows0, gs0).wait()
        didx(c0, d0, ds0).wait()
        pltpu.sync_copy(rows0, z_sh.at[d0], add=True)
        sidx(c0 + 2, s0, ss0).start()
        didx(c0 + 2, d0, ds0).start()

        gath(s1, rows1, gs1).wait()
        didx(c0 + 1, d1, ds1).wait()
        pltpu.sync_copy(rows1, z_sh.at[d1], add=True)
        sidx(c0 + 3, s1, ss1).start()
        didx(c0 + 3, d1, ds1).start()

        sidx(c0 + 2, s0, ss0).wait()
        gath(s0, rows0, gs0).start()
        sidx(c0 + 3, s1, ss1).wait()
        gath(s1, rows1, gs1).start()

    gath(s0, rows0, gs0).wait()
    didx(NCHUNK - 2, d0, ds0).wait()
    pltpu.sync_copy(rows0, z_sh.at[d0], add=True)
    gath(s1, rows1, gs1).wait()
    didx(NCHUNK - 1, d1, ds1).wait()
    pltpu.sync_copy(rows1, z_sh.at[d1], add=True)

    plsc.subcore_barrier()
    pltpu.sync_copy(z_sh.at[pl.ds(sid * RPT, RPT)],
                    out_hbm.at[cid, pl.ds(sid * RPT, RPT)])


BR = 400
GRID = N // BR


def _prep_body(x_ref, degp_ref, xs_ref, dinv_ref):
    deg = degp_ref[0] + degp_ref[1] + 1.0
    dinv = lax.rsqrt(deg)
    dinv_ref[...] = dinv
    xs_ref[...] = x_ref[...] * dinv


_prep = pl.pallas_call(
    _prep_body,
    grid=(GRID,),
    in_specs=[
        pl.BlockSpec((BR, D), lambda i: (i, 0)),
        pl.BlockSpec((NC, BR, 1), lambda i: (0, i, 0)),
    ],
    out_specs=[
        pl.BlockSpec((BR, D), lambda i: (i, 0)),
        pl.BlockSpec((BR, 1), lambda i: (i, 0)),
    ],
    out_shape=[
        jax.ShapeDtypeStruct((N, D), jnp.float32),
        jax.ShapeDtypeStruct((N, 1), jnp.float32),
    ],
)


def _epi_body(zp, xs, dinv, x,
              aw, ab, a1w, a1b, a2w, a2b, a3w, a3b,
              cw, cb, c1w, c1b, c2w, c2b, c3w, c3b,
              conc_ref, val_ref, csum):
    i = pl.program_id(0)
    f32 = jnp.float32
    z = zp[0] + zp[1]
    y = (z + xs[...]) * dinv[...]
    a = jnp.maximum(jnp.dot(y, aw[...], preferred_element_type=f32) + ab[...], 0.0)
    a = a + x[...]
    a = jnp.maximum(jnp.dot(a, a1w[...], preferred_element_type=f32) + a1b[...], 0.0)
    a = jnp.maximum(jnp.dot(a, a2w[...], preferred_element_type=f32) + a2b[...], 0.0)
    a3 = jnp.dot(a, a3w[...], preferred_element_type=f32) + a3b[...]
    conc_ref[...] = jax.nn.softplus(a3) + 1e-20

    c = jnp.maximum(jnp.dot(y, cw[...], preferred_element_type=f32) + cb[...], 0.0)
    c = c + x[...]

    @pl.when(i == 0)
    def _():
        csum[...] = jnp.zeros_like(csum)

    csum[...] += jnp.sum(c, axis=0, keepdims=True)

    @pl.when(i == pl.num_programs(0) - 1)
    def _():
        cv = csum[...]
        h = jnp.maximum(jnp.dot(cv, c1w[...], preferred_element_type=f32) + c1b[...], 0.0)
        h = jnp.maximum(jnp.dot(h, c2w[...], preferred_element_type=f32) + c2b[...], 0.0)
        val_ref[...] = jnp.dot(h, c3w[...], preferred_element_type=f32) + c3b[...]


def _full(shape):
    return pl.BlockSpec(shape, lambda i: tuple(0 for _ in shape))


_epi = pl.pallas_call(
    _epi_body,
    grid=(GRID,),
    in_specs=[
        pl.BlockSpec((NC, BR, D), lambda i: (0, i, 0)),
        pl.BlockSpec((BR, D), lambda i: (i, 0)),
        pl.BlockSpec((BR, 1), lambda i: (i, 0)),
        pl.BlockSpec((BR, D), lambda i: (i, 0)),
        _full((D, D)), _full((1, D)),
        _full((D, H)), _full((1, H)),
        _full((H, H)), _full((1, H)),
        _full((H, 1)), _full((1, 1)),
        _full((D, D)), _full((1, D)),
        _full((D, H)), _full((1, H)),
        _full((H, H)), _full((1, H)),
        _full((H, 1)), _full((1, 1)),
    ],
    out_specs=[
        pl.BlockSpec((BR, 1), lambda i: (i, 0)),
        pl.BlockSpec((1, 1), lambda i: (0, 0)),
    ],
    out_shape=[
        jax.ShapeDtypeStruct((N, 1), jnp.float32),
        jax.ShapeDtypeStruct((1, 1), jnp.float32),
    ],
    scratch_shapes=[pltpu.VMEM((1, D), jnp.float32)],
)


def kernel(x, edge_index,
           actor_conv_w, actor_conv_b, actor_l1_w, actor_l1_b,
           actor_l2_w, actor_l2_b, actor_l3_w, actor_l3_b,
           critic_conv_w, critic_conv_b, critic_l1_w, critic_l1_b,
           critic_l2_w, critic_l2_b, critic_l3_w, critic_l3_b):
    pad = E_PAD - E
    it = jnp.arange(pad, dtype=edge_index.dtype)
    src3 = jnp.concatenate(
        [edge_index[0], it % N]
    ).reshape(NW, NCHUNK, CHUNK)
    dst3 = jnp.concatenate(
        [edge_index[1], N + (it % (N_PAD - N))]
    ).reshape(NW, NCHUNK, CHUNK)

    degp = _deg_kernel(dst3).reshape(NC, N_PAD, 1)
    xs, dinv = _prep(x, degp)
    zp = _spmm_kernel(xs, src3, dst3)

    conc2d, val = _epi(
        zp, xs, dinv, x,
        actor_conv_w, actor_conv_b.reshape(1, D),
        actor_l1_w, actor_l1_b.reshape(1, H),
        actor_l2_w, actor_l2_b.reshape(1, H),
        actor_l3_w, actor_l3_b.reshape(1, 1),
        critic_conv_w, critic_conv_b.reshape(1, D),
        critic_l1_w, critic_l1_b.reshape(1, H),
        critic_l2_w, critic_l2_b.reshape(1, H),
        critic_l3_w, critic_l3_b.reshape(1, 1),
    )
    return conc2d.reshape(-1), val.reshape(-1)


# speedup vs baseline: 39.0631x; 1.0971x over previous
import dataclasses
import functools

import numpy as np

import jax
import jax.numpy as jnp
from jax import lax
from jax.experimental import pallas as pl
from jax.experimental.pallas import tpu as pltpu
from jax.experimental.pallas import tpu_sc as plsc

N = 10000
E = 320000
D = 128
H = 32

NC = 2
NS = 16
NW = NC * NS
CHUNK = 128
NCHUNK = 80
EPT = CHUNK * NCHUNK
E_PAD = EPT * NW
N_PAD = 10240
RPT = N_PAD // NS
HR = N_PAD // 128

_PAD_SRC = np.arange(E_PAD - E, dtype=np.int32) % N
_PAD_DST = N + np.arange(E_PAD - E, dtype=np.int32) % (N_PAD - N)

_mesh = plsc.VectorSubcoreMesh(core_axis_name="c", subcore_axis_name="s")

_cp = pltpu.CompilerParams()
if "needs_layout_passes" in pltpu.CompilerParams.__dataclass_fields__:
    _cp = dataclasses.replace(_cp, needs_layout_passes=False)


@functools.partial(
    pl.kernel,
    out_type=jax.ShapeDtypeStruct((NC, HR, 128), jnp.float32),
    mesh=_mesh,
    scratch_types=[
        pltpu.VMEM((CHUNK,), jnp.int32),
        pltpu.VMEM((CHUNK,), jnp.int32),
        pltpu.VMEM((HR, 128), jnp.float32),
        pltpu.VMEM((HR,), jnp.int32),
        pltpu.VMEM_SHARED((HR, 128), jnp.float32),
        pltpu.SemaphoreType.DMA,
        pltpu.SemaphoreType.DMA,
    ],
    compiler_params=_cp,
)
def _deg_kernel(dst_hbm, out_hbm, di0, di1, hist, rowidx, deg_sh, si0, si1):
    cid = lax.axis_index("c")
    sid = lax.axis_index("s")
    wid = cid * NS + sid
    i16 = lax.iota(jnp.int32, 16)
    ones = jnp.ones((16,), jnp.float32)

    @pl.loop(0, HR)
    def _(i):
        @pl.loop(0, 8)
        def _(j):
            hist[i, pl.ds(j * 16, 16)] = jnp.zeros((16,), jnp.float32)

    @pl.loop(0, HR // 16)
    def _(k):
        rowidx[pl.ds(k * 16, 16)] = i16 + k * 16

    @pl.when(sid == 0)
    def _():
        pltpu.sync_copy(hist, deg_sh)

    def idx_dma(c, dv, sem):
        return pltpu.make_async_copy(dst_hbm.at[wid, c], dv, sem)

    idx_dma(0, di0, si0).start()
    idx_dma(1, di1, si1).start()

    def accum(dv):
        @pl.loop(0, CHUNK // 16)
        def _(k):
            ids = dv[pl.ds(k * 16, 16)]
            row = lax.shift_right_logical(ids, 7)
            col = lax.bitwise_and(ids, 127)
            plsc.addupdate_scatter(hist, [row, col], ones)

    @pl.loop(0, NCHUNK // 2 - 1)
    def _(g):
        c0 = g * 2
        idx_dma(c0, di0, si0).wait()
        accum(di0)
        idx_dma(c0 + 2, di0, si0).start()
        idx_dma(c0 + 1, di1, si1).wait()
        accum(di1)
        idx_dma(c0 + 3, di1, si1).start()

    idx_dma(NCHUNK - 2, di0, si0).wait()
    accum(di0)
    idx_dma(NCHUNK - 1, di1, si1).wait()
    accum(di1)

    plsc.subcore_barrier()
    pltpu.sync_copy(hist, deg_sh.at[rowidx], add=True)
    plsc.subcore_barrier()

    @pl.when(sid < 10)
    def _():
        pltpu.sync_copy(deg_sh.at[pl.ds(sid * 8, 8)],
                        out_hbm.at[cid, pl.ds(sid * 8, 8)])


@functools.partial(
    pl.kernel,
    out_type=jax.ShapeDtypeStruct((NC, N_PAD, D), jnp.float32),
    mesh=_mesh,
    scratch_types=[
        pltpu.VMEM((CHUNK,), jnp.int32),
        pltpu.VMEM((CHUNK,), jnp.int32),
        pltpu.VMEM((CHUNK,), jnp.int32),
        pltpu.VMEM((CHUNK,), jnp.int32),
        pltpu.VMEM((CHUNK, D), jnp.float32),
        pltpu.VMEM((CHUNK, D), jnp.float32),
        pltpu.VMEM_SHARED((N_PAD, D), jnp.float32),
        pltpu.SemaphoreType.DMA,
        pltpu.SemaphoreType.DMA,
        pltpu.SemaphoreType.DMA,
        pltpu.SemaphoreType.DMA,
        pltpu.SemaphoreType.DMA,
        pltpu.SemaphoreType.DMA,
    ],
)
def _spmm_kernel(xs_hbm, src_hbm, dst_hbm, out_hbm,
                 s0, s1, d0, d1, rows0, rows1, z_sh,
                 ss0, ss1, ds0, ds1, gs0, gs1):
    cid = lax.axis_index("c")
    sid = lax.axis_index("s")
    wid = cid * NS + sid

    @pl.loop(0, CHUNK)
    def _(i):
        @pl.loop(0, D // 16)
        def _(j):
            rows0[i, pl.ds(j * 16, 16)] = jnp.zeros((16,), jnp.float32)

    @pl.loop(0, RPT // CHUNK)
    def _(k):
        pltpu.sync_copy(rows0, z_sh.at[pl.ds(sid * RPT + k * CHUNK, CHUNK)])

    plsc.subcore_barrier()

    def sidx(c, dv, sem):
        return pltpu.make_async_copy(src_hbm.at[wid, c], dv, sem)

    def didx(c, dv, sem):
        return pltpu.make_async_copy(dst_hbm.at[wid, c], dv, sem)

    def gath(sv, rows, sem):
        return pltpu.make_async_copy(xs_hbm.at[sv], rows, sem)

    sidx(0, s0, ss0).start()
    didx(0, d0, ds0).start()
    sidx(1, s1, ss1).start()
    didx(1, d1, ds1).start()
    sidx(0, s0, ss0).wait()
    gath(s0, rows0, gs0).start()
    sidx(1, s1, ss1).wait()
    gath(s1, rows1, gs1).start()

    @pl.loop(0, NCHUNK // 2 - 1)
    def _(g):
        c0 = g * 2
        gath(s0, rows0, gs0).wait()
        didx(c0, d0, ds0).wait()
        pltpu.sync_copy(rows0, z_sh.at[d0], add=True)
        sidx(c0 + 2, s0, ss0).start()
        didx(c0 + 2, d0, ds0).start()

        gath(s1, rows1, gs1).wait()
        didx(c0 + 1, d1, ds1).wait()
        pltpu.sync_copy(rows1, z_sh.at[d1], add=True)
        sidx(c0 + 3, s1, ss1).start()
        didx(c0 + 3, d1, ds1).start()

        sidx(c0 + 2, s0, ss0).wait()
        gath(s0, rows0, gs0).start()
        sidx(c0 + 3, s1, ss1).wait()
        gath(s1, rows1, gs1).start()

    gath(s0, rows0, gs0).wait()
    didx(NCHUNK - 2, d0, ds0).wait()
    pltpu.sync_copy(rows0, z_sh.at[d0], add=True)
    gath(s1, rows1, gs1).wait()
    didx(NCHUNK - 1, d1, ds1).wait()
    pltpu.sync_copy(rows1, z_sh.at[d1], add=True)

    plsc.subcore_barrier()
    pltpu.sync_copy(z_sh.at[pl.ds(sid * RPT, RPT)],
                    out_hbm.at[cid, pl.ds(sid * RPT, RPT)])


BR = 2000
GRID = N // BR


def _prep_body(x_ref, degp_ref, xs_ref, dinv_ref):
    deg = degp_ref[0] + degp_ref[1] + 1.0
    dinv = lax.rsqrt(deg)
    dinv_ref[...] = dinv
    xs_ref[...] = x_ref[...] * dinv


_prep = pl.pallas_call(
    _prep_body,
    grid=(GRID,),
    in_specs=[
        pl.BlockSpec((BR, D), lambda i: (i, 0)),
        pl.BlockSpec((NC, BR, 1), lambda i: (0, i, 0)),
    ],
    out_specs=[
        pl.BlockSpec((BR, D), lambda i: (i, 0)),
        pl.BlockSpec((BR, 1), lambda i: (i, 0)),
    ],
    out_shape=[
        jax.ShapeDtypeStruct((N, D), jnp.float32),
        jax.ShapeDtypeStruct((N, 1), jnp.float32),
    ],
)


def _epi_body(zp, xs, dinv, x,
              aw, ab, a1w, a1b, a2w, a2b, a3w, a3b,
              cw, cb, c1w, c1b, c2w, c2b, c3w, c3b,
              conc_ref, val_ref, csum):
    i = pl.program_id(0)
    f32 = jnp.float32
    z = zp[0] + zp[1]
    y = (z + xs[...]) * dinv[...]
    a = jnp.maximum(jnp.dot(y, aw[...], preferred_element_type=f32) + ab[...], 0.0)
    a = a + x[...]
    a = jnp.maximum(jnp.dot(a, a1w[...], preferred_element_type=f32) + a1b[...], 0.0)
    a = jnp.maximum(jnp.dot(a, a2w[...], preferred_element_type=f32) + a2b[...], 0.0)
    a3 = jnp.dot(a, a3w[...], preferred_element_type=f32) + a3b[...]
    conc_ref[...] = jax.nn.softplus(a3) + 1e-20

    c = jnp.maximum(jnp.dot(y, cw[...], preferred_element_type=f32) + cb[...], 0.0)
    c = c + x[...]

    @pl.when(i == 0)
    def _():
        csum[...] = jnp.zeros_like(csum)

    csum[...] += jnp.sum(c, axis=0, keepdims=True)

    @pl.when(i == pl.num_programs(0) - 1)
    def _():
        cv = csum[...]
        h = jnp.maximum(jnp.dot(cv, c1w[...], preferred_element_type=f32) + c1b[...], 0.0)
        h = jnp.maximum(jnp.dot(h, c2w[...], preferred_element_type=f32) + c2b[...], 0.0)
        val_ref[...] = jnp.dot(h, c3w[...], preferred_element_type=f32) + c3b[...]


def _full(shape):
    return pl.BlockSpec(shape, lambda i: tuple(0 for _ in shape))


_epi = pl.pallas_call(
    _epi_body,
    grid=(GRID,),
    in_specs=[
        pl.BlockSpec((NC, BR, D), lambda i: (0, i, 0)),
        pl.BlockSpec((BR, D), lambda i: (i, 0)),
        pl.BlockSpec((BR, 1), lambda i: (i, 0)),
        pl.BlockSpec((BR, D), lambda i: (i, 0)),
        _full((D, D)), _full((1, D)),
        _full((D, H)), _full((1, H)),
        _full((H, H)), _full((1, H)),
        _full((H, 1)), _full((1, 1)),
        _full((D, D)), _full((1, D)),
        _full((D, H)), _full((1, H)),
        _full((H, H)), _full((1, H)),
        _full((H, 1)), _full((1, 1)),
    ],
    out_specs=[
        pl.BlockSpec((BR, 1), lambda i: (i, 0)),
        pl.BlockSpec((1, 1), lambda i: (0, 0)),
    ],
    out_shape=[
        jax.ShapeDtypeStruct((N, 1), jnp.float32),
        jax.ShapeDtypeStruct((1, 1), jnp.float32),
    ],
    scratch_shapes=[pltpu.VMEM((1, D), jnp.float32)],
)


def kernel(x, edge_index,
           actor_conv_w, actor_conv_b, actor_l1_w, actor_l1_b,
           actor_l2_w, actor_l2_b, actor_l3_w, actor_l3_b,
           critic_conv_w, critic_conv_b, critic_l1_w, critic_l1_b,
           critic_l2_w, critic_l2_b, critic_l3_w, critic_l3_b):
    src3 = jnp.concatenate(
        [edge_index[0], jnp.asarray(_PAD_SRC)]
    ).reshape(NW, NCHUNK, CHUNK)
    dst3 = jnp.concatenate(
        [edge_index[1], jnp.asarray(_PAD_DST)]
    ).reshape(NW, NCHUNK, CHUNK)

    degp = _deg_kernel(dst3).reshape(NC, N_PAD, 1)
    xs, dinv = _prep(x, degp)
    zp = _spmm_kernel(xs, src3, dst3)

    conc2d, val = _epi(
        zp, xs, dinv, x,
        actor_conv_w, actor_conv_b.reshape(1, D),
        actor_l1_w, actor_l1_b.reshape(1, H),
        actor_l2_w, actor_l2_b.reshape(1, H),
        actor_l3_w, actor_l3_b.reshape(1, 1),
        critic_conv_w, critic_conv_b.reshape(1, D),
        critic_l1_w, critic_l1_b.reshape(1, H),
        critic_l2_w, critic_l2_b.reshape(1, H),
        critic_l3_w, critic_l3_b.reshape(1, 1),
    )
    return conc2d.reshape(-1), val.reshape(-1)
